# baseline (speedup 1.0000x reference)
import functools

import jax
import jax.numpy as jnp
from jax import lax
from jax.experimental import pallas as pl
from jax.experimental.pallas import tpu as pltpu
from jax.experimental.pallas import tpu_sc as plsc

BATCH = 16384
D = 32
NT = 4
NC = 2
NS = 16
NW = NC * NS
BPW = BATCH // NW
CHUNK = 128
NCHUNK = BPW // CHUNK
L = 16
GROUPS = BPW // L
SW = (NT - 1) * D

_MESH = plsc.VectorSubcoreMesh(core_axis_name="c", subcore_axis_name="s")
_PARAMS = pltpu.CompilerParams(
    use_tc_tiling_on_sc=False, needs_layout_passes=False)


@functools.partial(
    pl.kernel,
    out_type=jax.ShapeDtypeStruct((BATCH, SW), jnp.float32),
    mesh=_MESH,
    compiler_params=_PARAMS,
    scratch_types=[
        pltpu.VMEM((NT - 1, BPW), jnp.int32),
        pltpu.VMEM((2 * L, D), jnp.float32),
        pltpu.VMEM((2 * L, D), jnp.float32),
        pltpu.VMEM((2 * L, D), jnp.float32),
        pltpu.VMEM((BPW, SW), jnp.float32),
        pltpu.SemaphoreType.DMA,
    ],
)
def _small_kernel(g_idx, a_idx, o_idx, w_gender, w_age, w_occ, out_hbm,
                  idx_v, sg_v, sa_v, so_v, big_v, isem):
    wid = lax.axis_index("s") * NC + lax.axis_index("c")
    base = wid * BPW
    stage = [pltpu.async_copy(arr.at[pl.ds(base, BPW)], idx_v.at[t], isem)
             for t, arr in enumerate((g_idx, a_idx, o_idx))]
    for w, buf in ((w_gender, sg_v), (w_age, sa_v), (w_occ, so_v)):
        stage.append(pltpu.async_copy(
            w, buf.at[pl.ds(0, w.shape[0])], isem))
    for c in stage:
        c.wait()

    def group_body(i, carry):
        rbase = i * L
        for t, buf in ((0, sg_v), (1, sa_v), (2, so_v)):
            ridx = idx_v[t, pl.ds(rbase, L)]
            for j in range(L):
                s = ridx[j]
                r = rbase + j
                for h in (0, L):
                    big_v[r, pl.ds(t * D + h, L)] = buf[s, pl.ds(h, L)]
        return carry

    lax.fori_loop(0, GROUPS, group_body, 0)
    pltpu.sync_copy(big_v, out_hbm.at[pl.ds(base, BPW)])


@functools.partial(
    pl.kernel,
    out_type=jax.ShapeDtypeStruct((BATCH, NT * D), jnp.float32),
    mesh=_MESH,
    compiler_params=_PARAMS,
    scratch_types=[
        pltpu.VMEM((BPW,), jnp.int32),
        pltpu.VMEM((BPW, D), jnp.float32),
        pltpu.VMEM((BPW, SW), jnp.float32),
        pltpu.SemaphoreType.DMA,
        pltpu.SemaphoreType.DMA,
    ],
)
def _area_kernel(z_idx, w_area, small_hbm, out_hbm,
                 idx_v, area_v, sm_v, gsem, osem):
    wid = lax.axis_index("s") * NC + lax.axis_index("c")
    base = wid * BPW
    pltpu.sync_copy(z_idx.at[pl.ds(base, BPW)], idx_v)
    copies = [
        pltpu.async_copy(
            w_area.at[idx_v.at[pl.ds(j * CHUNK, CHUNK)]],
            area_v.at[pl.ds(j * CHUNK, CHUNK)],
            gsem,
        )
        for j in range(NCHUNK)
    ]
    pltpu.sync_copy(small_hbm.at[pl.ds(base, BPW)], sm_v)
    wr_small = pltpu.async_copy(
        sm_v, out_hbm.at[pl.ds(base, BPW), pl.ds(0, SW)], osem)
    for c in copies:
        c.wait()
    wr_area = pltpu.async_copy(
        area_v, out_hbm.at[pl.ds(base, BPW), pl.ds(SW, D)], osem)
    wr_small.wait()
    wr_area.wait()


def kernel(gender_idx, age_idx, occupation_idx, area_idx,
           W_gender, W_age, W_occupation, W_area):
    small = _small_kernel(
        gender_idx.astype(jnp.int32), age_idx.astype(jnp.int32),
        occupation_idx.astype(jnp.int32), W_gender, W_age, W_occupation)
    w_area_lin = lax.optimization_barrier(
        W_area.reshape(-1)).reshape(W_area.shape)
    return _area_kernel(area_idx.astype(jnp.int32), w_area_lin, small)


# speedup vs baseline: 1.0039x; 1.0039x over previous
import functools

import jax
import jax.numpy as jnp
from jax import lax
from jax.experimental import pallas as pl
from jax.experimental.pallas import tpu as pltpu
from jax.experimental.pallas import tpu_sc as plsc

BATCH = 16384
D = 32
NT = 4
NC = 2
NS = 16
NW = NC * NS
BPW = BATCH // NW
CHUNK = 128
NCHUNK = BPW // CHUNK
L = 16
GROUPS = BPW // L
SW = (NT - 1) * D

_MESH = plsc.VectorSubcoreMesh(core_axis_name="c", subcore_axis_name="s")
_PARAMS = pltpu.CompilerParams(
    use_tc_tiling_on_sc=False, needs_layout_passes=False)


@functools.partial(
    pl.kernel,
    out_type=jax.ShapeDtypeStruct((BATCH, SW), jnp.float32),
    mesh=_MESH,
    compiler_params=_PARAMS,
    scratch_types=[
        pltpu.VMEM((NT - 1, BPW), jnp.int32),
        pltpu.VMEM((2 * L, D), jnp.float32),
        pltpu.VMEM((2 * L, D), jnp.float32),
        pltpu.VMEM((2 * L, D), jnp.float32),
        pltpu.VMEM((BPW, SW), jnp.float32),
        pltpu.SemaphoreType.DMA,
    ],
)
def _small_kernel(g_idx, a_idx, o_idx, w_gender, w_age, w_occ, out_hbm,
                  idx_v, sg_v, sa_v, so_v, big_v, isem):
    wid = lax.axis_index("s") * NC + lax.axis_index("c")
    base = wid * BPW
    stage = [pltpu.async_copy(arr.at[pl.ds(base, BPW)], idx_v.at[t], isem)
             for t, arr in enumerate((g_idx, a_idx, o_idx))]
    for w, buf in ((w_gender, sg_v), (w_age, sa_v), (w_occ, so_v)):
        stage.append(pltpu.async_copy(
            w, buf.at[pl.ds(0, w.shape[0])], isem))
    for c in stage:
        c.wait()

    def group_body(i, carry):
        rbase = i * L
        for t, buf in ((0, sg_v), (1, sa_v), (2, so_v)):
            ridx = idx_v[t, pl.ds(rbase, L)]
            for j in range(L):
                s = ridx[j]
                r = rbase + j
                for h in (0, L):
                    big_v[r, pl.ds(t * D + h, L)] = buf[s, pl.ds(h, L)]
        return carry

    lax.fori_loop(0, GROUPS, group_body, 0)
    pltpu.sync_copy(big_v, out_hbm.at[pl.ds(base, BPW)])


@functools.partial(
    pl.kernel,
    out_type=jax.ShapeDtypeStruct((BATCH, NT * D), jnp.float32),
    mesh=_MESH,
    compiler_params=_PARAMS,
    scratch_types=[
        pltpu.VMEM((BPW,), jnp.int32),
        pltpu.VMEM((BPW, D), jnp.float32),
        pltpu.VMEM((BPW, SW), jnp.float32),
        pltpu.SemaphoreType.DMA,
        pltpu.SemaphoreType.DMA,
    ],
)
def _area_kernel(z_idx, w_area, small_hbm, out_hbm,
                 idx_v, area_v, sm_v, gsem, osem):
    wid = lax.axis_index("s") * NC + lax.axis_index("c")
    base = wid * BPW
    sm_cp = pltpu.async_copy(small_hbm.at[pl.ds(base, BPW)], sm_v, osem)
    pltpu.sync_copy(z_idx.at[pl.ds(base, BPW)], idx_v)
    copies = [
        pltpu.async_copy(
            w_area.at[idx_v.at[pl.ds(j * CHUNK, CHUNK)]],
            area_v.at[pl.ds(j * CHUNK, CHUNK)],
            gsem,
        )
        for j in range(NCHUNK)
    ]
    sm_cp.wait()
    wr_small = pltpu.async_copy(
        sm_v, out_hbm.at[pl.ds(base, BPW), pl.ds(0, SW)], osem)
    for c in copies:
        c.wait()
    wr_area = pltpu.async_copy(
        area_v, out_hbm.at[pl.ds(base, BPW), pl.ds(SW, D)], osem)
    wr_small.wait()
    wr_area.wait()


def kernel(gender_idx, age_idx, occupation_idx, area_idx,
           W_gender, W_age, W_occupation, W_area):
    small = _small_kernel(
        gender_idx.astype(jnp.int32), age_idx.astype(jnp.int32),
        occupation_idx.astype(jnp.int32), W_gender, W_age, W_occupation)
    return _area_kernel(area_idx.astype(jnp.int32), W_area, small)
